# baseline (speedup 1.0000x reference)
import functools

import jax
import jax.numpy as jnp
from jax import lax
from jax.experimental import pallas as pl
from jax.experimental.pallas import tpu as pltpu
from jax.experimental.pallas import tpu_sc as plsc

_DELAY = 8388608
_FEEDBACK = 0.5
_NW = 32
_CHUNK = _DELAY // _NW

_mesh = plsc.VectorSubcoreMesh(core_axis_name="c", subcore_axis_name="s")


@functools.partial(
    pl.kernel,
    mesh=_mesh,
    out_type=[
        jax.ShapeDtypeStruct((1,), jnp.float32),
        jax.ShapeDtypeStruct((_DELAY,), jnp.float32),
    ],
    scratch_types=[
        pltpu.VMEM((16,), jnp.int32),
        pltpu.VMEM((16,), jnp.float32),
        pltpu.VMEM((16,), jnp.float32),
        pltpu.VMEM((16,), jnp.float32),
    ],
    compiler_params=pltpu.CompilerParams(needs_layout_passes=False),
)
def _sc_kernel(x_hbm, idx_hbm, buf_hbm, outs_hbm, outb_hbm, ivm, xvm, svm, bvm):
    wid = lax.axis_index("s") * 2 + lax.axis_index("c")
    base = wid * _CHUNK
    pltpu.sync_copy(buf_hbm.at[pl.ds(base, _CHUNK)], outb_hbm.at[pl.ds(base, _CHUNK)])
    pltpu.sync_copy(idx_hbm, ivm.at[pl.ds(0, 1)])
    idxs = ivm[...][0]
    own = (idxs >= base) & (idxs < base + _CHUNK)

    @pl.when(own)
    def _update():
        lane = lax.iota(jnp.int32, 16)
        pltpu.sync_copy(x_hbm, xvm.at[pl.ds(0, 1)])
        xs = xvm[...][0]
        aligned = (idxs // 16) * 16
        off = idxs - aligned
        pltpu.sync_copy(buf_hbm.at[pl.ds(aligned, 16)], bvm)
        offv = jnp.full((16,), off, jnp.int32)
        bo = plsc.load_gather(bvm, [offv])[0]
        svm[...] = jnp.where(lane == 0, -xs + bo, 0.0)
        pltpu.sync_copy(svm.at[pl.ds(0, 1)], outs_hbm)
        newv = jnp.full((16,), xs + bo * _FEEDBACK, jnp.float32)
        plsc.store_scatter(bvm, [offv], newv, mask=lane == 0)
        pltpu.sync_copy(bvm, outb_hbm.at[pl.ds(aligned, 16)])


def kernel(x, buffer, buffer_index):
    idx = jnp.asarray(buffer_index, jnp.int32).reshape(1)
    xs = x.reshape(1).astype(jnp.float32)
    out_s, out_buf = _sc_kernel(xs, idx, buffer)
    return (out_s[0], out_buf)
